# baseline (speedup 1.0000x reference)
import functools

import jax
import jax.numpy as jnp
from jax import lax
from jax.experimental import pallas as pl
from jax.experimental.pallas import tpu as pltpu
from jax.experimental.pallas import tpu_sc as plsc

N = 4096
NMOL = 16
MAXP = 32768
CUT2 = 25.0
NC, NS, L = 2, 16, 16
NW = NC * NS
NGRP = N // L
GPW = NGRP // NW
WP = MAXP // NW

@functools.cache
def _mesh():
    return plsc.VectorSubcoreMesh(
        core_axis_name="c", subcore_axis_name="s",
        num_cores=NC, num_subcores=NS)


def _wid():
    return lax.axis_index("s") * NC + lax.axis_index("c")


def _seg_starts(b_v):
    m = lax.iota(jnp.int32, L)
    lo = jnp.zeros((L,), jnp.int32)
    hi = jnp.full((L,), N, jnp.int32)

    def step(_, carry):
        lo, hi = carry
        mid = lax.div(lo + hi, 2)
        bm = plsc.load_gather(b_v, [mid])
        lt = bm < m
        return jnp.where(lt, mid + 1, lo), jnp.where(lt, hi, mid)

    lo, hi = lax.fori_loop(0, 12, step, (lo, hi))
    return lo


def _sqrt16(d2):
    d2c = jnp.maximum(d2, 1e-30)
    i = plsc.bitcast(d2c, jnp.int32)
    y = plsc.bitcast(0x5F3759DF - (i >> 1), jnp.float32)
    for _ in range(3):
        y = y * (1.5 - 0.5 * d2c * y * y)
    return d2c * y


def _count_body(x_h, y_h, z_h, b_h, cnt_h,
                x_v, y_v, z_v, b_v, seg_v, cg_v):
    wid = _wid()
    pltpu.sync_copy(x_h, x_v)
    pltpu.sync_copy(y_h, y_v)
    pltpu.sync_copy(z_h, z_v)
    pltpu.sync_copy(b_h, b_v)
    seg_v[...] = _seg_starts(b_v)
    iot = lax.iota(jnp.int32, L)

    def group(gi, _):
        g = wid + gi * NW
        i0 = g * L
        bg = b_v[pl.ds(i0, L)]
        sg = plsc.load_gather(seg_v, [bg])
        xg = x_v[pl.ds(i0, L)]
        yg = y_v[pl.ds(i0, L)]
        zg = z_v[pl.ds(i0, L)]
        cvec = jnp.zeros((L,), jnp.int32)
        for l in range(L):
            i = i0 + l
            si = sg[l]
            xi, yi, zi = xg[l], yg[l], zg[l]
            jb0 = (si // L) * L
            nch = lax.div(i - jb0 + (L - 1), L)

            def chunk(c, acc, jb0=jb0, jv0=jb0 + iot, si=si, i=i,
                      xi=xi, yi=yi, zi=zi):
                jb = jb0 + c * L
                jv = jv0 + c * L
                dx = x_v[pl.ds(jb, L)] - xi
                dy = y_v[pl.ds(jb, L)] - yi
                dz = z_v[pl.ds(jb, L)] - zi
                d2 = dx * dx + dy * dy + dz * dz
                pred = (jv >= si) & (jv < i) & (d2 < CUT2)
                return acc + pred.astype(jnp.int32)

            acc = lax.fori_loop(0, nch, chunk, jnp.zeros((L,), jnp.int32))
            cvec = jnp.where(iot == l, jnp.sum(acc), cvec)
        cg_v[...] = cvec
        pltpu.sync_copy(cg_v, cnt_h.at[pl.ds(i0, L)])
        return 0

    lax.fori_loop(0, GPW, group, 0)


@functools.cache
def _count_call():
  return pl.kernel(
    _count_body,
    out_type=jax.ShapeDtypeStruct((N,), jnp.int32),
    mesh=_mesh(),
    compiler_params=pltpu.CompilerParams(needs_layout_passes=False),
    scratch_types=[
        pltpu.VMEM((N,), jnp.float32),
        pltpu.VMEM((N,), jnp.float32),
        pltpu.VMEM((N,), jnp.float32),
        pltpu.VMEM((N,), jnp.int32),
        pltpu.VMEM((L,), jnp.int32),
        pltpu.VMEM((L,), jnp.int32),
    ],
  )


def _scan_body(c_ref, b_ref, o_ref, rs_ref):
    x = c_ref[...]
    s = x
    for k in (1, 2, 4, 8, 16, 32, 64):
        s = s + jnp.concatenate(
            [jnp.zeros((32, k), jnp.int32), s[:, :-k]], axis=1)
    rt = s[:, 127:128]
    r = rt
    for k in (1, 2, 4, 8, 16):
        r = r + jnp.concatenate(
            [jnp.zeros((k, 1), jnp.int32), r[:-k, :]], axis=0)
    o_ref[...] = s - x + (r - rt)

    b = b_ref[...]
    prev_last = jnp.concatenate(
        [jnp.full((1, 1), -1, jnp.int32), b[:-1, 127:128]], axis=0)
    bp = jnp.concatenate([prev_last, b[:, :-1]], axis=1)
    gi = lax.broadcasted_iota(jnp.int32, (32, 128), 0)
    ki = lax.broadcasted_iota(jnp.int32, (32, 128), 1)
    seed = jnp.where(b != bp, gi * 128 + ki, 0)
    m = seed
    for k in (1, 2, 4, 8, 16, 32, 64):
        m = jnp.maximum(m, jnp.concatenate(
            [jnp.zeros((32, k), jnp.int32), m[:, :-k]], axis=1))
    rm = jnp.concatenate(
        [jnp.zeros((1, 1), jnp.int32), m[:-1, 127:128]], axis=0)
    for k in (1, 2, 4, 8, 16):
        rm = jnp.maximum(rm, jnp.concatenate(
            [jnp.zeros((k, 1), jnp.int32), rm[:-k, :]], axis=0))
    rs_ref[...] = jnp.maximum(m, rm)


def _excl_scan(counts, b):
    off, rs = pl.pallas_call(
        _scan_body,
        out_shape=(jax.ShapeDtypeStruct((32, 128), jnp.int32),
                   jax.ShapeDtypeStruct((32, 128), jnp.int32)),
    )(counts.reshape(32, 128), b.reshape(32, 128))
    return off.reshape(-1), rs.reshape(-1)


def _emit_body(x_h, y_h, z_h, rs_h, off_h,
               ni_h, nj_h, d_h, dx_h, dy_h, dz_h,
               x_v, y_v, z_v, rs_v, off_v,
               ni_s, nj_s, d_s, dx_s, dy_s, dz_s):
    wid = _wid()
    pltpu.sync_copy(x_h, x_v)
    pltpu.sync_copy(y_h, y_v)
    pltpu.sync_copy(z_h, z_v)
    pltpu.sync_copy(rs_h, rs_v)
    pltpu.sync_copy(off_h, off_v)
    iot = lax.iota(jnp.int32, L)
    ws = wid * WP
    we = ws + WP

    def fill(k, _):
        ni_s[pl.ds(k * L, L)] = jnp.full((L,), -1, jnp.int32)
        nj_s[pl.ds(k * L, L)] = jnp.full((L,), -1, jnp.int32)
        d_s[pl.ds(k * L, L)] = jnp.zeros((L,), jnp.float32)
        dx_s[pl.ds(k * L, L)] = jnp.zeros((L,), jnp.float32)
        dy_s[pl.ds(k * L, L)] = jnp.zeros((L,), jnp.float32)
        dz_s[pl.ds(k * L, L)] = jnp.zeros((L,), jnp.float32)
        return 0

    lax.fori_loop(0, WP // L, fill, 0)

    def _at(ref, idx):
        return plsc.load_gather(ref, [jnp.full((L,), idx, jnp.int32)])[0]

    def bstep(_, carry):
        lo, hi = carry
        mid = lax.div(lo + hi, 2)
        gt = _at(off_v, mid) > ws
        return jnp.where(gt, lo, mid + 1), jnp.where(gt, mid, hi)

    lo, _ = lax.fori_loop(0, 12, bstep, (jnp.int32(0), jnp.int32(N)))
    r0 = jnp.maximum(lo - 1, 0)

    def cond(carry):
        r, rank = carry
        return (r < N) & (rank < we)

    def body(carry):
        r, rank0 = carry
        rv = jnp.full((L,), r, jnp.int32)
        si = plsc.load_gather(rs_v, [rv])[0]
        xi = plsc.load_gather(x_v, [rv])
        yi = plsc.load_gather(y_v, [rv])
        zi = plsc.load_gather(z_v, [rv])
        jb0 = (si // L) * L
        nch = lax.div(r - jb0 + (L - 1), L)

        def chunk(c, rank):
            jb = jb0 + c * L
            jv = jb + iot
            dx = xi - x_v[pl.ds(jb, L)]
            dy = yi - y_v[pl.ds(jb, L)]
            dz = zi - z_v[pl.ds(jb, L)]
            d2 = dx * dx + dy * dy + dz * dz
            pred = (jv >= si) & (jv < r) & (d2 < CUT2)
            pc = plsc.all_reduce_population_count(pred)[0]

            @pl.when((pc > 0) & (rank + pc > ws) & (rank < we))
            def _():
                inc = plsc.cumsum(pred.astype(jnp.int32))
                pos = rank + inc - 1
                wm = pred & (pos >= ws) & (pos < we)
                loc = pos - ws
                plsc.store_scatter(ni_s, [loc],
                                   jnp.full((L,), r, jnp.int32), mask=wm)
                plsc.store_scatter(nj_s, [loc], jv, mask=wm)
                plsc.store_scatter(d_s, [loc], _sqrt16(d2), mask=wm)
                plsc.store_scatter(dx_s, [loc], dx, mask=wm)
                plsc.store_scatter(dy_s, [loc], dy, mask=wm)
                plsc.store_scatter(dz_s, [loc], dz, mask=wm)

            return rank + pc

        rank1 = lax.fori_loop(0, nch, chunk, rank0)
        return r + 1, rank1

    lax.while_loop(cond, body, (r0, _at(off_v, r0)))

    pltpu.sync_copy(ni_s, ni_h.at[pl.ds(ws, WP)])
    pltpu.sync_copy(nj_s, nj_h.at[pl.ds(ws, WP)])
    pltpu.sync_copy(d_s, d_h.at[pl.ds(ws, WP)])
    pltpu.sync_copy(dx_s, dx_h.at[pl.ds(ws, WP)])
    pltpu.sync_copy(dy_s, dy_h.at[pl.ds(ws, WP)])
    pltpu.sync_copy(dz_s, dz_h.at[pl.ds(ws, WP)])


_f32 = jnp.float32


@functools.cache
def _emit_call():
  return pl.kernel(
    _emit_body,
    out_type=(
        jax.ShapeDtypeStruct((MAXP,), jnp.int32),
        jax.ShapeDtypeStruct((MAXP,), jnp.int32),
        jax.ShapeDtypeStruct((MAXP,), _f32),
        jax.ShapeDtypeStruct((MAXP,), _f32),
        jax.ShapeDtypeStruct((MAXP,), _f32),
        jax.ShapeDtypeStruct((MAXP,), _f32),
    ),
    mesh=_mesh(),
    compiler_params=pltpu.CompilerParams(needs_layout_passes=False),
    scratch_types=[
        pltpu.VMEM((N,), _f32),
        pltpu.VMEM((N,), _f32),
        pltpu.VMEM((N,), _f32),
        pltpu.VMEM((N,), jnp.int32),
        pltpu.VMEM((N,), jnp.int32),
        pltpu.VMEM((WP,), jnp.int32),
        pltpu.VMEM((WP,), jnp.int32),
        pltpu.VMEM((WP,), _f32),
        pltpu.VMEM((WP,), _f32),
        pltpu.VMEM((WP,), _f32),
        pltpu.VMEM((WP,), _f32),
    ],
  )


def kernel(pos, batch):
    b = batch.astype(jnp.int32)
    x = pos[:, 0]
    y = pos[:, 1]
    z = pos[:, 2]
    counts = _count_call()(x, y, z, b)
    off, rs = _excl_scan(counts, b)
    ni, nj, d, dx, dy, dz = _emit_call()(x, y, z, rs, off)
    neighbors = jnp.stack([ni, nj], axis=0)
    distance_vecs = jnp.stack([dx, dy, dz], axis=1)
    return neighbors, d, distance_vecs


# speedup vs baseline: 1.2286x; 1.2286x over previous
import functools

import jax
import jax.numpy as jnp
from jax import lax
from jax.experimental import pallas as pl
from jax.experimental.pallas import tpu as pltpu
from jax.experimental.pallas import tpu_sc as plsc

N = 4096
NMOL = 16
MAXP = 32768
CUT2 = 25.0
NC, NS, L = 2, 16, 16
NW = NC * NS
NGRP = N // L
GPW = NGRP // NW
WP = MAXP // NW

@functools.cache
def _mesh():
    return plsc.VectorSubcoreMesh(
        core_axis_name="c", subcore_axis_name="s",
        num_cores=NC, num_subcores=NS)


def _wid():
    return lax.axis_index("s") * NC + lax.axis_index("c")


def _seg_starts(b_v):
    m = lax.iota(jnp.int32, L)
    lo = jnp.zeros((L,), jnp.int32)
    hi = jnp.full((L,), N, jnp.int32)

    def step(_, carry):
        lo, hi = carry
        mid = lax.div(lo + hi, 2)
        bm = plsc.load_gather(b_v, [mid])
        lt = bm < m
        return jnp.where(lt, mid + 1, lo), jnp.where(lt, hi, mid)

    lo, hi = lax.fori_loop(0, 12, step, (lo, hi))
    return lo


def _sqrt16(d2):
    d2c = jnp.maximum(d2, 1e-30)
    i = plsc.bitcast(d2c, jnp.int32)
    y = plsc.bitcast(0x5F3759DF - (i >> 1), jnp.float32)
    for _ in range(3):
        y = y * (1.5 - 0.5 * d2c * y * y)
    return d2c * y


def _count_body(x_h, y_h, z_h, b_h, cnt_h,
                x_v, y_v, z_v, b_v, seg_v, cg_v):
    wid = _wid()
    pltpu.sync_copy(x_h, x_v)
    pltpu.sync_copy(y_h, y_v)
    pltpu.sync_copy(z_h, z_v)
    pltpu.sync_copy(b_h, b_v)
    seg_v[...] = _seg_starts(b_v)
    iot = lax.iota(jnp.int32, L)

    def group(gi, _):
        g = wid + gi * NW
        i0 = g * L
        bg = b_v[pl.ds(i0, L)]
        sg = plsc.load_gather(seg_v, [bg])
        xg = x_v[pl.ds(i0, L)]
        yg = y_v[pl.ds(i0, L)]
        zg = z_v[pl.ds(i0, L)]
        cvec = jnp.zeros((L,), jnp.int32)
        for l in range(L):
            i = i0 + l
            si = sg[l]
            xi, yi, zi = xg[l], yg[l], zg[l]
            jb0 = (si // L) * L
            nch = lax.div(i - jb0 + (L - 1), L)

            def chunk(c, acc, jb0=jb0, jv0=jb0 + iot, si=si, i=i,
                      xi=xi, yi=yi, zi=zi):
                jb = jb0 + c * L
                jv = jv0 + c * L
                dx = x_v[pl.ds(jb, L)] - xi
                dy = y_v[pl.ds(jb, L)] - yi
                dz = z_v[pl.ds(jb, L)] - zi
                d2 = dx * dx + dy * dy + dz * dz
                pred = (jv >= si) & (jv < i) & (d2 < CUT2)
                return acc + pred.astype(jnp.int32)

            acc = lax.fori_loop(0, nch, chunk, jnp.zeros((L,), jnp.int32))
            cvec = jnp.where(iot == l, jnp.sum(acc), cvec)
        cg_v[...] = cvec
        pltpu.sync_copy(cg_v, cnt_h.at[pl.ds(i0, L)])
        return 0

    lax.fori_loop(0, GPW, group, 0)


@functools.cache
def _count_call():
  return pl.kernel(
    _count_body,
    out_type=jax.ShapeDtypeStruct((N,), jnp.int32),
    mesh=_mesh(),
    compiler_params=pltpu.CompilerParams(needs_layout_passes=False),
    scratch_types=[
        pltpu.VMEM((N,), jnp.float32),
        pltpu.VMEM((N,), jnp.float32),
        pltpu.VMEM((N,), jnp.float32),
        pltpu.VMEM((N,), jnp.int32),
        pltpu.VMEM((L,), jnp.int32),
        pltpu.VMEM((L,), jnp.int32),
    ],
  )


def _scan_body(c_ref, b_ref, o_ref, rs_ref):
    x = c_ref[...]
    s = x
    for k in (1, 2, 4, 8, 16, 32, 64):
        s = s + jnp.concatenate(
            [jnp.zeros((32, k), jnp.int32), s[:, :-k]], axis=1)
    rt = s[:, 127:128]
    r = rt
    for k in (1, 2, 4, 8, 16):
        r = r + jnp.concatenate(
            [jnp.zeros((k, 1), jnp.int32), r[:-k, :]], axis=0)
    o_ref[...] = s - x + (r - rt)

    b = b_ref[...]
    prev_last = jnp.concatenate(
        [jnp.full((1, 1), -1, jnp.int32), b[:-1, 127:128]], axis=0)
    bp = jnp.concatenate([prev_last, b[:, :-1]], axis=1)
    gi = lax.broadcasted_iota(jnp.int32, (32, 128), 0)
    ki = lax.broadcasted_iota(jnp.int32, (32, 128), 1)
    seed = jnp.where(b != bp, gi * 128 + ki, 0)
    m = seed
    for k in (1, 2, 4, 8, 16, 32, 64):
        m = jnp.maximum(m, jnp.concatenate(
            [jnp.zeros((32, k), jnp.int32), m[:, :-k]], axis=1))
    rm = jnp.concatenate(
        [jnp.zeros((1, 1), jnp.int32), m[:-1, 127:128]], axis=0)
    for k in (1, 2, 4, 8, 16):
        rm = jnp.maximum(rm, jnp.concatenate(
            [jnp.zeros((k, 1), jnp.int32), rm[:-k, :]], axis=0))
    rs_ref[...] = jnp.maximum(m, rm)


def _excl_scan(counts, b):
    off, rs = pl.pallas_call(
        _scan_body,
        out_shape=(jax.ShapeDtypeStruct((32, 128), jnp.int32),
                   jax.ShapeDtypeStruct((32, 128), jnp.int32)),
    )(counts.reshape(32, 128), b.reshape(32, 128))
    return off.reshape(-1), rs.reshape(-1)


def _emit_body(x_h, y_h, z_h, rs_h, off_h,
               ni_h, nj_h, d_h, dx_h, dy_h, dz_h,
               x_v, y_v, z_v, rs_v, off_v,
               ni_s, nj_s, d_s, dx_s, dy_s, dz_s):
    wid = _wid()
    pltpu.sync_copy(x_h, x_v)
    pltpu.sync_copy(y_h, y_v)
    pltpu.sync_copy(z_h, z_v)
    pltpu.sync_copy(rs_h, rs_v)
    pltpu.sync_copy(off_h, off_v)
    iot = lax.iota(jnp.int32, L)
    ws = wid * WP
    we = ws + WP

    def fill(k, _):
        ni_s[pl.ds(k * L, L)] = jnp.full((L,), -1, jnp.int32)
        nj_s[pl.ds(k * L, L)] = jnp.full((L,), -1, jnp.int32)
        d_s[pl.ds(k * L, L)] = jnp.zeros((L,), jnp.float32)
        dx_s[pl.ds(k * L, L)] = jnp.zeros((L,), jnp.float32)
        dy_s[pl.ds(k * L, L)] = jnp.zeros((L,), jnp.float32)
        dz_s[pl.ds(k * L, L)] = jnp.zeros((L,), jnp.float32)
        return 0

    lax.fori_loop(0, WP // L, fill, 0)

    def _at(ref, idx):
        return plsc.load_gather(ref, [jnp.full((L,), idx, jnp.int32)])[0]

    def bstep(_, carry):
        lo, hi = carry
        mid = lax.div(lo + hi, 2)
        gt = _at(off_v, mid) > ws
        return jnp.where(gt, lo, mid + 1), jnp.where(gt, mid, hi)

    lo, _ = lax.fori_loop(0, 12, bstep, (jnp.int32(0), jnp.int32(N)))
    r0 = jnp.maximum(lo - 1, 0)

    def cond(carry):
        r, rank = carry
        return (r < N) & (rank < we)

    def body(carry):
        r, rank0 = carry
        rv = jnp.full((L,), r, jnp.int32)
        si = plsc.load_gather(rs_v, [rv])[0]
        xi = plsc.load_gather(x_v, [rv])
        yi = plsc.load_gather(y_v, [rv])
        zi = plsc.load_gather(z_v, [rv])
        jb0 = (si // L) * L
        nch = lax.div(r - jb0 + (L - 1), L)

        def chunk(c, rank):
            jb = jb0 + c * L
            jv = jb + iot
            dx = xi - x_v[pl.ds(jb, L)]
            dy = yi - y_v[pl.ds(jb, L)]
            dz = zi - z_v[pl.ds(jb, L)]
            d2 = dx * dx + dy * dy + dz * dz
            pred = (jv >= si) & (jv < r) & (d2 < CUT2)
            inc = plsc.cumsum(pred.astype(jnp.int32))
            pos = rank + inc - 1
            wm = pred & (pos >= ws) & (pos < we)
            loc = pos - ws
            plsc.store_scatter(ni_s, [loc],
                               jnp.full((L,), r, jnp.int32), mask=wm)
            plsc.store_scatter(nj_s, [loc], jv, mask=wm)
            plsc.store_scatter(d_s, [loc], _sqrt16(d2), mask=wm)
            plsc.store_scatter(dx_s, [loc], dx, mask=wm)
            plsc.store_scatter(dy_s, [loc], dy, mask=wm)
            plsc.store_scatter(dz_s, [loc], dz, mask=wm)
            return rank + inc[L - 1]

        rank1 = lax.fori_loop(0, nch, chunk, rank0)
        return r + 1, rank1

    lax.while_loop(cond, body, (r0, _at(off_v, r0)))

    pltpu.sync_copy(ni_s, ni_h.at[pl.ds(ws, WP)])
    pltpu.sync_copy(nj_s, nj_h.at[pl.ds(ws, WP)])
    pltpu.sync_copy(d_s, d_h.at[pl.ds(ws, WP)])
    pltpu.sync_copy(dx_s, dx_h.at[pl.ds(ws, WP)])
    pltpu.sync_copy(dy_s, dy_h.at[pl.ds(ws, WP)])
    pltpu.sync_copy(dz_s, dz_h.at[pl.ds(ws, WP)])


_f32 = jnp.float32


@functools.cache
def _emit_call():
  return pl.kernel(
    _emit_body,
    out_type=(
        jax.ShapeDtypeStruct((MAXP,), jnp.int32),
        jax.ShapeDtypeStruct((MAXP,), jnp.int32),
        jax.ShapeDtypeStruct((MAXP,), _f32),
        jax.ShapeDtypeStruct((MAXP,), _f32),
        jax.ShapeDtypeStruct((MAXP,), _f32),
        jax.ShapeDtypeStruct((MAXP,), _f32),
    ),
    mesh=_mesh(),
    compiler_params=pltpu.CompilerParams(needs_layout_passes=False),
    scratch_types=[
        pltpu.VMEM((N,), _f32),
        pltpu.VMEM((N,), _f32),
        pltpu.VMEM((N,), _f32),
        pltpu.VMEM((N,), jnp.int32),
        pltpu.VMEM((N,), jnp.int32),
        pltpu.VMEM((WP,), jnp.int32),
        pltpu.VMEM((WP,), jnp.int32),
        pltpu.VMEM((WP,), _f32),
        pltpu.VMEM((WP,), _f32),
        pltpu.VMEM((WP,), _f32),
        pltpu.VMEM((WP,), _f32),
    ],
  )


def kernel(pos, batch):
    b = batch.astype(jnp.int32)
    x = pos[:, 0]
    y = pos[:, 1]
    z = pos[:, 2]
    counts = _count_call()(x, y, z, b)
    off, rs = _excl_scan(counts, b)
    ni, nj, d, dx, dy, dz = _emit_call()(x, y, z, rs, off)
    neighbors = jnp.stack([ni, nj], axis=0)
    distance_vecs = jnp.stack([dx, dy, dz], axis=1)
    return neighbors, d, distance_vecs
